# baseline (speedup 1.0000x reference)
import functools

import jax
import jax.numpy as jnp
from jax import lax
from jax.experimental import pallas as pl
from jax.experimental.pallas import tpu as pltpu
from jax.experimental.pallas import tpu_sc as plsc

VOCAB = 1000000
DIM = 32
BATCH = 16384

NC = 2
NS = 16
NW = NC * NS
B_PER_W = BATCH // NW
ROWS_PER_W = 2 * B_PER_W
GCHUNK = 128
NCHUNK = ROWS_PER_W // GCHUNK
NBLK = B_PER_W // 16


def _make_kernel():
  mesh = plsc.VectorSubcoreMesh(core_axis_name="c", subcore_axis_name="s")

  @functools.partial(
      pl.kernel,
      mesh=mesh,
      out_type=jax.ShapeDtypeStruct((BATCH,), jnp.float32),
      compiler_params=pltpu.CompilerParams(
          needs_layout_passes=False, use_tc_tiling_on_sc=False
      ),
      scratch_types=[
          pltpu.VMEM((NCHUNK, GCHUNK), jnp.int32),
          pltpu.VMEM((ROWS_PER_W, DIM), jnp.float32),
          pltpu.VMEM((B_PER_W,), jnp.float32),
          pltpu.SemaphoreType.DMA,
      ],
  )
  def k(xs_hbm, table_hbm, out_hbm, idx_v, rows_v, out_v, sem):
    wid = lax.axis_index("s") * NC + lax.axis_index("c")

    pltpu.sync_copy(xs_hbm.at[wid], idx_v)

    copies = []
    for j in range(NCHUNK):
      copies.append(
          pltpu.async_copy(
              table_hbm.at[idx_v.at[j]],
              rows_v.at[pl.ds(j * GCHUNK, GCHUNK), :],
              sem,
          )
      )
    for c in copies:
      c.wait()

    lane = lax.iota(jnp.int32, 16)
    half = jnp.full((16,), 0.5, jnp.float32)
    three_half = jnp.full((16,), 1.5, jnp.float32)
    one = jnp.full((16,), 1.0, jnp.float32)
    magic = jnp.full((16,), 0x5F3759DF, jnp.int32)

    def block(g, carry):
      row_a = (g * 16 + lane) * 2
      row_b = row_a + 1
      acc_d = jnp.zeros((16,), jnp.float32)
      acc_a = jnp.zeros((16,), jnp.float32)
      acc_b = jnp.zeros((16,), jnp.float32)
      for d in range(DIM):
        col = (lane + d) & (DIM - 1)
        va = plsc.load_gather(rows_v, [row_a, col])
        vb = plsc.load_gather(rows_v, [row_b, col])
        acc_d = acc_d + va * vb
        acc_a = acc_a + va * va
        acc_b = acc_b + vb * vb
      x = acc_a * acc_b
      y = plsc.bitcast(magic - (plsc.bitcast(x, jnp.int32) >> 1), jnp.float32)
      hx = half * x
      for _ in range(3):
        y = y * (three_half - hx * y * y)
      cos = acc_d * y
      sig = one / (one + jnp.exp(-cos))
      out_v[pl.ds(g * 16, 16)] = (sig + one) * half
      return carry

    lax.fori_loop(0, NBLK, block, 0)

    pltpu.sync_copy(out_v, out_hbm.at[pl.ds(wid * B_PER_W, B_PER_W)])

  return k


_kernel = _make_kernel()


def kernel(xs, table):
  xs3 = xs.reshape(NW, NCHUNK, GCHUNK)
  return _kernel(xs3, table)


# speedup vs baseline: 2.0611x; 2.0611x over previous
import functools

import jax
import jax.numpy as jnp
from jax import lax
from jax.experimental import pallas as pl
from jax.experimental.pallas import tpu as pltpu
from jax.experimental.pallas import tpu_sc as plsc
from jax._src.pallas.mosaic import sc_core as _sc_core

VOCAB = 1000000
DIM = 32
BATCH = 16384
NSLOT = 2 * BATCH

NC = 2
NS = 16
NW = NC * NS
NTILES = (VOCAB + 127) // 128
TPW = NTILES // NW
TAIL = NTILES - NW * TPW
EXT = 128


def _make_gather_kernel():
  mesh = plsc.VectorSubcoreMesh(core_axis_name="c", subcore_axis_name="s")

  @functools.partial(
      pl.kernel,
      mesh=mesh,
      compiler_params=pltpu.CompilerParams(needs_layout_passes=False),
      out_type=jax.ShapeDtypeStruct((NSLOT, 128), jnp.float32),
      scratch_types=[
          pltpu.VMEM((NSLOT // 128, 128), jnp.int32),
          pltpu.VMEM((NSLOT + 16,), jnp.int32),
          pltpu.VMEM((NSLOT + 16,), jnp.int32),
          pltpu.VMEM((2, 4, 8, 128), jnp.float32),
          pltpu.VMEM((EXT, 128), jnp.float32),
          pltpu.VMEM((EXT,), jnp.int32),
          pltpu.SMEM((256,), jnp.int32),
          pltpu.SMEM((256,), jnp.int32),
          pltpu.SMEM((256,), jnp.int32),
          pltpu.SemaphoreType.DMA,
          pltpu.SemaphoreType.DMA,
          pltpu.SemaphoreType.DMA,
      ],
  )
  def k(xs_hbm, t3_hbm, st_hbm, xsv, hit, srt, ring, exb, exs,
        cnt, cur, st0, sem0, sem1, sem2):
    wid = lax.axis_index("s") * NC + lax.axis_index("c")
    t0 = wid * TPW
    tcnt = jnp.where(wid == NW - 1, TPW + TAIL, TPW)
    lane_lo = t0 * 128

    lanev = lax.iota(jnp.int32, 16)
    ones16 = jnp.ones((16,), jnp.int32)
    neg16 = jnp.full((16,), -1, jnp.int32)

    pltpu.sync_copy(xs_hbm, xsv)
    lim = tcnt * 128

    def scan_row(r, off):
      def scan_vec(c, off):
        idx = xsv[r, pl.ds(c * 16, 16)]
        rl = idx - lane_lo
        m = (rl >= 0) & (rl < lim)
        slot = (r * 128 + c * 16) + lanev
        packed = (rl << 16) + slot
        plsc.store_compressed(hit.at[pl.ds(off, 16)], packed, mask=m)
        npop = lax.reduce_sum(jnp.where(m, ones16, 0), (0,))
        return off + npop

      return lax.fori_loop(0, 8, scan_vec, off)

    nh = lax.fori_loop(0, NSLOT // 128, scan_row, jnp.int32(0))

    def zero_cnt(t, carry):
      cnt[t] = 0
      return carry

    lax.fori_loop(0, 256, zero_cnt, 0)

    def count_one(h, carry):
      v = hit[pl.ds(h, 16)][0]
      t = v >> 23
      cnt[t] = cnt[t] + 1
      return carry

    lax.fori_loop(0, nh, count_one, 0)

    def prefix(t, acc):
      c = cnt[t]
      cur[t] = acc
      st0[t] = acc
      return acc + c

    lax.fori_loop(0, 256, prefix, jnp.int32(0))

    def place_one(h, carry):
      v = hit[pl.ds(h, 16)][0]
      t = v >> 23
      p = cur[t]
      cur[t] = p + 1
      plsc.store_scatter(
          srt, [jnp.full((16,), p, jnp.int32)],
          jnp.full((16,), v, jnp.int32), mask=lanev == 0,
      )
      return carry

    lax.fori_loop(0, nh, place_one, 0)

    sems = (sem0, sem1)

    def fire(c, slot):
      start = lane_lo + c * 128
      start = pl.multiple_of(start, 128)
      for i in range(4):
        pltpu.async_copy(
            t3_hbm.at[i, :, pl.ds(start, 128)], ring.at[slot, i], sems[slot]
        )

    def drain(slot):
      for i in range(4):
        pltpu.make_async_copy(
            t3_hbm.at[i, :, pl.ds(0, 128)], ring.at[slot, i], sems[slot]
        ).wait()

    def reset_exs():
      for i in range(EXT // 16):
        exs[pl.ds(i * 16, 16)] = neg16

    reset_exs()

    def exflush():
      pltpu.async_copy(
          exb, st_hbm.at[_sc_core.Indices(exs, ignored_value=-1)], sem2
      )
      pltpu.make_async_copy(exb, st_hbm.at[pl.ds(0, EXT)], sem2).wait()
      reset_exs()

    dlo = lanev // 8
    dls = lanev % 8
    dhi = 2 + dlo

    def make_process(slot):
      def process(c, ex):
        end_k = cur[c]
        k0 = st0[c]

        def one_hit(j, ex):
          v = srt[pl.ds(k0 + j, 16)][0]
          l = (v >> 16) & 127
          s = v & 0xFFFF
          er = lax.rem(ex, jnp.int32(EXT))
          lsp = jnp.full((16,), l, jnp.int32)
          slotv = jnp.full((16,), slot, jnp.int32)
          ga = plsc.load_gather(ring, [slotv, dlo, dls, lsp])
          gb = plsc.load_gather(ring, [slotv, dhi, dls, lsp])
          exb[er, pl.ds(0, 16)] = ga
          exb[er, pl.ds(16, 16)] = gb
          plsc.store_scatter(
              exs, [jnp.full((16,), er, jnp.int32)],
              jnp.full((16,), s, jnp.int32), mask=lanev == 0,
          )
          ex = ex + 1
          pl.when(lax.rem(ex, jnp.int32(EXT)) == 0)(exflush)
          return ex

        return lax.fori_loop(0, end_k - k0, one_hit, ex)

      return process

    process0 = make_process(0)
    process1 = make_process(1)

    fire(0, 0)

    def step(c2, ex):
      c = c2 * 2
      pl.when(c + 1 < tcnt)(lambda: fire(c + 1, 1))
      drain(0)
      ex = process0(c, ex)
      pl.when(c + 2 < tcnt)(lambda: fire(c + 2, 0))

      def do_odd(ex):
        drain(1)
        return process1(c + 1, ex)

      ex = lax.cond(c + 1 < tcnt, do_odd, lambda e: e, ex)
      return ex

    nsteps = (TPW + TAIL + 1) // 2

    def guarded_step(c2, ex):
      return lax.cond(c2 * 2 < tcnt, lambda e: step(c2, e), lambda e: e, ex)

    ex = lax.fori_loop(0, nsteps, guarded_step, jnp.int32(0))
    pl.when(lax.rem(ex, jnp.int32(EXT)) != 0)(exflush)

  return k


_gather = _make_gather_kernel()


def _compute_body(x_ref, o_ref):
  x = x_ref[...]
  a = x[:, 0, :]
  b = x[:, 1, :]
  mask = lax.broadcasted_iota(jnp.int32, a.shape, 1) < DIM
  zero = jnp.zeros_like(a)
  ab = jnp.where(mask, a * b, zero)
  aa = jnp.where(mask, a * a, zero)
  bb = jnp.where(mask, b * b, zero)
  dot = jnp.sum(ab, axis=1)
  na = jnp.sum(aa, axis=1)
  nb = jnp.sum(bb, axis=1)
  cos = dot * lax.rsqrt(na * nb)
  sig = 1.0 / (1.0 + jnp.exp(-cos))
  o_ref[...] = (sig + 1.0) * 0.5


_BLK = 2048


def _compute(staging):
  x = staging.reshape(BATCH, 2, 128)
  return pl.pallas_call(
      _compute_body,
      grid=(BATCH // _BLK,),
      in_specs=[pl.BlockSpec((_BLK, 2, 128), lambda i: (i, 0, 0))],
      out_specs=pl.BlockSpec((_BLK,), lambda i: (i,)),
      out_shape=jax.ShapeDtypeStruct((BATCH,), jnp.float32),
  )(x)


def kernel(xs, table):
  t3 = table.T.reshape(4, 8, VOCAB)
  xs2 = xs.reshape(NSLOT // 128, 128)
  staging = _gather(xs2, t3)
  return _compute(staging)


# speedup vs baseline: 2.1885x; 1.0618x over previous
import functools

import jax
import jax.numpy as jnp
from jax import lax
from jax.experimental import pallas as pl
from jax.experimental.pallas import tpu as pltpu
from jax.experimental.pallas import tpu_sc as plsc
from jax._src.pallas.mosaic import sc_core as _sc_core

VOCAB = 1000000
DIM = 32
BATCH = 16384
NSLOT = 2 * BATCH

NC = 2
NS = 16
NW = NC * NS
NTILES = (VOCAB + 127) // 128
TPW = NTILES // NW
TAIL = NTILES - NW * TPW
FL = 64


def _make_gather_kernel():
  mesh = plsc.VectorSubcoreMesh(core_axis_name="c", subcore_axis_name="s")

  @functools.partial(
      pl.kernel,
      mesh=mesh,
      compiler_params=pltpu.CompilerParams(needs_layout_passes=False),
      out_type=jax.ShapeDtypeStruct((NSLOT, 128), jnp.float32),
      scratch_types=[
          pltpu.VMEM((NSLOT // 128, 128), jnp.int32),
          pltpu.VMEM((NSLOT + 16,), jnp.int32),
          pltpu.VMEM((NSLOT + 16,), jnp.int32),
          pltpu.VMEM((2, 4, 8, 128), jnp.float32),
          pltpu.VMEM((2 * FL, 128), jnp.float32),
          pltpu.VMEM((2 * FL,), jnp.int32),
          pltpu.SMEM((256,), jnp.int32),
          pltpu.SMEM((256,), jnp.int32),
          pltpu.SMEM((256,), jnp.int32),
          pltpu.SemaphoreType.DMA,
          pltpu.SemaphoreType.DMA,
          pltpu.SemaphoreType.DMA,
      ],
  )
  def k(xs_hbm, t3_hbm, st_hbm, xsv, hit, srt, ring, exb, exs,
        cnt, cur, st0, sem0, sem1, sem2):
    wid = lax.axis_index("s") * NC + lax.axis_index("c")
    t0 = wid * TPW
    tcnt = jnp.where(wid == NW - 1, TPW + TAIL, TPW)
    lane_lo = t0 * 128

    lanev = lax.iota(jnp.int32, 16)
    neg16 = jnp.full((16,), -1, jnp.int32)
    slot_lane = (lanev & 1) * BATCH + (lanev >> 1)

    pltpu.sync_copy(xs_hbm, xsv)
    lim = tcnt * 128

    def scan_row(r, off):
      base = r * 128
      for c in range(8):
        idx = xsv[r, pl.ds(c * 16, 16)]
        rl = idx - lane_lo
        m = (rl >= 0) & (rl < lim)
        slot = ((base + c * 16) >> 1) + slot_lane
        packed = (rl << 16) + slot
        plsc.store_compressed(hit.at[pl.ds(off, 16)], packed, mask=m)
        pc = plsc.all_reduce_population_count(m)
        off = off + pc[0]
      return off

    nh = lax.fori_loop(0, NSLOT // 128, scan_row, jnp.int32(0))

    def zero_cnt(t, carry):
      cnt[t] = 0
      return carry

    lax.fori_loop(0, 256, zero_cnt, 0)

    def count_one(h, carry):
      v = hit[pl.ds(h, 16)][0]
      t = v >> 23
      cnt[t] = cnt[t] + 1
      return carry

    lax.fori_loop(0, nh, count_one, 0)

    def prefix(t, acc):
      c = cnt[t]
      cur[t] = acc
      st0[t] = acc
      return acc + c

    lax.fori_loop(0, 256, prefix, jnp.int32(0))

    def place_one(h, carry):
      v = hit[pl.ds(h, 16)][0]
      t = v >> 23
      p = cur[t]
      cur[t] = p + 1
      plsc.store_scatter(
          srt, [jnp.full((16,), p, jnp.int32)],
          jnp.full((16,), v, jnp.int32), mask=lanev == 0,
      )
      return carry

    lax.fori_loop(0, nh, place_one, 0)

    sems = (sem0, sem1)

    def fire(c, slot):
      start = lane_lo + c * 128
      start = pl.multiple_of(start, 128)
      for i in range(4):
        pltpu.async_copy(
            t3_hbm.at[i, :, pl.ds(start, 128)], ring.at[slot, i], sems[slot]
        )

    def drain(slot):
      for i in range(4):
        pltpu.make_async_copy(
            t3_hbm.at[i, :, pl.ds(0, 128)], ring.at[slot, i], sems[slot]
        ).wait()

    for i in range(2 * FL // 16):
      exs[pl.ds(i * 16, 16)] = neg16

    def wait_one_flush():
      pltpu.make_async_copy(
          exb.at[pl.ds(0, FL), :], st_hbm.at[pl.ds(0, FL)], sem2
      ).wait()

    def flush(half, ex):
      base = half * FL
      other = (1 - half) * FL
      pl.when(ex >= 2 * FL)(wait_one_flush)
      pltpu.async_copy(
          exb.at[pl.ds(base, FL), :],
          st_hbm.at[_sc_core.Indices(
              exs.at[pl.ds(base, FL)], ignored_value=-1)],
          sem2,
      )
      for i in range(FL // 16):
        exs[pl.ds(other + i * 16, 16)] = neg16

    dlo = lanev // 8
    dls = lanev % 8
    dhi = 2 + dlo

    def make_process(slot):
      slotv = jnp.full((16,), slot, jnp.int32)

      def process(c, ex):
        end_k = cur[c]
        k0 = st0[c]

        def one_hit(j, ex):
          v = srt[pl.ds(k0 + j, 16)][0]
          l = (v >> 16) & 127
          s = v & 0xFFFF
          er = lax.rem(ex, jnp.int32(2 * FL))
          lsp = jnp.full((16,), l, jnp.int32)
          ga = plsc.load_gather(ring, [slotv, dlo, dls, lsp])
          gb = plsc.load_gather(ring, [slotv, dhi, dls, lsp])
          exb[er, pl.ds(0, 16)] = ga
          exb[er, pl.ds(16, 16)] = gb
          plsc.store_scatter(
              exs, [jnp.full((16,), er, jnp.int32)],
              jnp.full((16,), s, jnp.int32), mask=lanev == 0,
          )
          ex = ex + 1
          er2 = lax.rem(ex, jnp.int32(2 * FL))
          pl.when(er2 == FL)(lambda: flush(0, ex))
          pl.when(er2 == 0)(lambda: flush(1, ex))
          return ex

        return lax.fori_loop(0, end_k - k0, one_hit, ex)

      return process

    process0 = make_process(0)
    process1 = make_process(1)

    def step(c, ex):
      pl.when(c + 1 < tcnt)(lambda: fire(c + 1, 1))
      drain(0)
      ex = process0(c, ex)
      pl.when(c + 2 < tcnt)(lambda: fire(c + 2, 0))

      def do_odd(ex):
        drain(1)
        return process1(c + 1, ex)

      return lax.cond(c + 1 < tcnt, do_odd, lambda e: e, ex)

    fire(0, 0)
    ex = lax.fori_loop(0, TPW // 2, lambda c2, ex: step(c2 * 2, ex),
                       jnp.int32(0))

    def tail_steps():
      e = ex
      for c in range(TPW, TPW + TAIL + 1, 2):
        e = lax.cond(c < tcnt, lambda e, c=c: step(c, e), lambda e: e, e)
      return e

    ex = lax.cond(wid == NW - 1, tail_steps, lambda: ex)

    er2 = lax.rem(ex, jnp.int32(2 * FL))
    in_lo = lax.rem(ex, jnp.int32(FL)) != 0
    nfull = ex // FL
    pl.when(nfull >= 1)(wait_one_flush)
    pl.when(in_lo & (er2 < FL))(lambda: flush(0, jnp.int32(0)))
    pl.when(in_lo & (er2 > FL))(lambda: flush(1, jnp.int32(0)))
    pl.when(in_lo)(wait_one_flush)

  return k


_gather = _make_gather_kernel()


def _compute_body(a_ref, b_ref, o_ref):
  a = a_ref[...]
  b = b_ref[...]
  mask = lax.broadcasted_iota(jnp.int32, a.shape, 1) < DIM
  zero = jnp.zeros_like(a)
  ab = jnp.where(mask, a * b, zero)
  aa = jnp.where(mask, a * a, zero)
  bb = jnp.where(mask, b * b, zero)
  dot = jnp.sum(ab, axis=1)
  na = jnp.sum(aa, axis=1)
  nb = jnp.sum(bb, axis=1)
  cos = dot * lax.rsqrt(na * nb)
  sig = 1.0 / (1.0 + jnp.exp(-cos))
  o_ref[...] = (sig + 1.0) * 0.5


_BLK = 4096


def _compute(staging):
  nblk = BATCH // _BLK
  return pl.pallas_call(
      _compute_body,
      grid=(nblk,),
      in_specs=[
          pl.BlockSpec((_BLK, 128), lambda i: (i, 0)),
          pl.BlockSpec((_BLK, 128), lambda i, n=nblk: (i + n, 0)),
      ],
      out_specs=pl.BlockSpec((_BLK,), lambda i: (i,)),
      out_shape=jax.ShapeDtypeStruct((BATCH,), jnp.float32),
  )(staging, staging)


def kernel(xs, table):
  t3 = table.T.reshape(4, 8, VOCAB)
  xs2 = xs.reshape(NSLOT // 128, 128)
  staging = _gather(xs2, t3)
  return _compute(staging)


# speedup vs baseline: 2.4765x; 1.1316x over previous
import functools

import jax
import jax.numpy as jnp
from jax import lax
from jax.experimental import pallas as pl
from jax.experimental.pallas import tpu as pltpu
from jax.experimental.pallas import tpu_sc as plsc
from jax._src.pallas.mosaic import sc_core as _sc_core

VOCAB = 1000000
DIM = 32
BATCH = 16384
NSLOT = 2 * BATCH

NC = 2
NS = 16
NW = NC * NS
NTILES = (VOCAB + 127) // 128
TPW = NTILES // NW
TAIL = NTILES - NW * TPW
FL = 64
_RB = 1


def _make_gather_kernel():
  mesh = plsc.VectorSubcoreMesh(core_axis_name="c", subcore_axis_name="s")

  @functools.partial(
      pl.kernel,
      mesh=mesh,
      compiler_params=pltpu.CompilerParams(needs_layout_passes=False),
      out_type=jax.ShapeDtypeStruct((NSLOT, 128), jnp.float32),
      scratch_types=[
          pltpu.VMEM((NSLOT // 128, 128), jnp.int32),
          pltpu.VMEM((NSLOT + 16,), jnp.int32),
          pltpu.VMEM((NSLOT + 16,), jnp.int32),
          pltpu.VMEM((2, 4, 8, 128), jnp.float32),
          pltpu.VMEM((2 * FL, 128), jnp.float32),
          pltpu.VMEM((2 * FL,), jnp.int32),
          pltpu.VMEM((272,), jnp.int32),
          pltpu.VMEM((272,), jnp.int32),
          pltpu.VMEM((272,), jnp.int32),
          pltpu.SemaphoreType.DMA,
          pltpu.SemaphoreType.DMA,
          pltpu.SemaphoreType.DMA,
      ],
  )
  def k(xs_hbm, t3_hbm, st_hbm, xsv, hit, srt, ring, exb, exs,
        cntv, curv, st0v, sem0, sem1, sem2):
    wid = lax.axis_index("s") * NC + lax.axis_index("c")
    t0 = wid * TPW
    tcnt = jnp.where(wid == NW - 1, TPW + TAIL, TPW)
    lane_lo = t0 * 128

    lanev = lax.iota(jnp.int32, 16)
    neg16 = jnp.full((16,), -1, jnp.int32)
    slot_lane = (lanev & 1) * BATCH + (lanev >> 1)

    pltpu.sync_copy(xs_hbm, xsv)
    lim = tcnt * 128

    def scan_row(r, off):
      base = r * 128
      for c in range(8):
        idx = xsv[r, pl.ds(c * 16, 16)]
        rl = idx - lane_lo
        m = (rl >= 0) & (rl < lim)
        slot = ((base + c * 16) >> 1) + slot_lane
        packed = (rl << 16) + slot
        plsc.store_compressed(hit.at[pl.ds(off, 16)], packed, mask=m)
        pc = plsc.all_reduce_population_count(m)
        off = off + pc[0]
      return off

    nh = lax.fori_loop(0, NSLOT // 128, scan_row, jnp.int32(0))

    zero16 = jnp.zeros((16,), jnp.int32)
    for i in range(16):
      cntv[pl.ds(i * 16, 16)] = zero16

    ngrp = (nh + 15) >> 4

    def count_grp(g, carry):
      m = g * 16 + lanev < nh
      v = hit[pl.ds(g * 16, 16)]
      t = (v >> 23) & 255
      rank, lastm = plsc.scan_count(t, mask=m)
      plsc.addupdate_scatter(
          cntv, [t], rank - _RB + 1, mask=m & lastm
      )
      return carry

    lax.fori_loop(0, ngrp, count_grp, 0)

    carry = jnp.int32(0)
    for i in range(16):
      cv = cntv[pl.ds(i * 16, 16)]
      inc = plsc.cumsum(cv) + carry
      st0v[pl.ds(i * 16, 16)] = inc - cv
      curv[pl.ds(i * 16, 16)] = inc - cv
      carry = inc[15]

    def place_grp(g, carry):
      m = g * 16 + lanev < nh
      v = hit[pl.ds(g * 16, 16)]
      t = (v >> 23) & 255
      rank, lastm = plsc.scan_count(t, mask=m)
      base = plsc.load_gather(curv, [t])
      pos = base + (rank - _RB)
      plsc.store_scatter(srt, [pos], v, mask=m)
      plsc.addupdate_scatter(
          curv, [t], rank - _RB + 1, mask=m & lastm
      )
      return carry

    lax.fori_loop(0, ngrp, place_grp, 0)

    sems = (sem0, sem1)

    def fire(c, slot):
      start = lane_lo + c * 128
      start = pl.multiple_of(start, 128)
      for i in range(4):
        pltpu.async_copy(
            t3_hbm.at[i, :, pl.ds(start, 128)], ring.at[slot, i], sems[slot]
        )

    def drain(slot):
      for i in range(4):
        pltpu.make_async_copy(
            t3_hbm.at[i, :, pl.ds(0, 128)], ring.at[slot, i], sems[slot]
        ).wait()

    for i in range(2 * FL // 16):
      exs[pl.ds(i * 16, 16)] = neg16

    def wait_one_flush():
      pltpu.make_async_copy(
          exb.at[pl.ds(0, FL), :], st_hbm.at[pl.ds(0, FL)], sem2
      ).wait()

    def fire_flush(half):
      base = half * FL
      pltpu.async_copy(
          exb.at[pl.ds(base, FL), :],
          st_hbm.at[_sc_core.Indices(
              exs.at[pl.ds(base, FL)], ignored_value=-1)],
          sem2,
      )

    def reset_half(half):
      base = half * FL
      for i in range(FL // 16):
        exs[pl.ds(base + i * 16, 16)] = neg16

    dlo = lanev // 8
    dls = lanev % 8
    dhi = 2 + dlo

    def make_process(slot):
      slotv = jnp.full((16,), slot, jnp.int32)

      def process(c, ex):
        k0 = st0v[pl.ds(c, 16)][0]
        kcnt = cntv[pl.ds(c, 16)][0]

        def grp(g, ex):
          n = jnp.minimum(kcnt - g * 16, 16)
          m = lanev < n
          v = srt[pl.ds(k0 + g * 16, 16)]
          l = (v >> 16) & 127
          s = v & 0xFFFF
          erv = lax.rem(ex + lanev, jnp.int32(2 * FL))
          r = lax.rem(ex, jnp.int32(FL))
          cross = (r + n) >= FL
          half_done = lax.rem(ex // FL, jnp.int32(2))
          ex_new = ex + n
          pl.when(cross & (ex_new >= 2 * FL))(wait_one_flush)
          pl.when(cross & (half_done == 1))(lambda: reset_half(0))
          pl.when(cross & (half_done == 0))(lambda: reset_half(1))
          for d in range(DIM):
            gv = plsc.load_gather(
                ring,
                [slotv, jnp.full((16,), d // 8, jnp.int32),
                 jnp.full((16,), d % 8, jnp.int32), l],
            )
            plsc.store_scatter(
                exb, [erv, jnp.full((16,), d, jnp.int32)], gv, mask=m
            )
          n_pre = jnp.minimum(n, FL - r)
          m_pre = lanev < n_pre
          m_post = m & ~m_pre
          plsc.store_scatter(exs, [erv], s, mask=m_pre)
          pl.when(cross & (half_done == 0))(lambda: fire_flush(0))
          pl.when(cross & (half_done == 1))(lambda: fire_flush(1))
          plsc.store_scatter(exs, [erv], s, mask=m_post)
          return ex_new

        return lax.fori_loop(0, (kcnt + 15) >> 4, grp, ex)

      return process

    process0 = make_process(0)
    process1 = make_process(1)

    def step(c, ex):
      pl.when(c + 1 < tcnt)(lambda: fire(c + 1, 1))
      drain(0)
      ex = process0(c, ex)
      pl.when(c + 2 < tcnt)(lambda: fire(c + 2, 0))

      def do_odd(ex):
        drain(1)
        return process1(c + 1, ex)

      return lax.cond(c + 1 < tcnt, do_odd, lambda e: e, ex)

    fire(0, 0)
    ex = lax.fori_loop(0, TPW // 2, lambda c2, ex: step(c2 * 2, ex),
                       jnp.int32(0))

    def tail_steps():
      e = ex
      for c in range(TPW, TPW + TAIL + 1, 2):
        e = lax.cond(c < tcnt, lambda e, c=c: step(c, e), lambda e: e, e)
      return e

    ex = lax.cond(wid == NW - 1, tail_steps, lambda: ex)

    nfull = ex // FL
    partial = lax.rem(ex, jnp.int32(FL)) != 0
    ph = lax.rem(nfull, jnp.int32(2))
    pl.when(nfull >= 1)(wait_one_flush)
    pl.when(partial & (ph == 0))(lambda: fire_flush(0))
    pl.when(partial & (ph == 1))(lambda: fire_flush(1))
    pl.when(partial)(wait_one_flush)

  return k


_gather = _make_gather_kernel()


def _compute_body(a_ref, b_ref, o_ref):
  a = a_ref[...]
  b = b_ref[...]
  mask = lax.broadcasted_iota(jnp.int32, a.shape, 1) < DIM
  zero = jnp.zeros_like(a)
  dot = jnp.sum(jnp.where(mask, a * b, zero), axis=1)
  na = jnp.sum(jnp.where(mask, a * a, zero), axis=1)
  nb = jnp.sum(jnp.where(mask, b * b, zero), axis=1)
  cos = dot * lax.rsqrt(na * nb)
  sig = 1.0 / (1.0 + jnp.exp(-cos))
  o_ref[...] = (sig + 1.0) * 0.5


_BLK = 4096


def _compute(staging):
  nblk = BATCH // _BLK
  return pl.pallas_call(
      _compute_body,
      grid=(nblk,),
      in_specs=[
          pl.BlockSpec((_BLK, 128), lambda i: (i, 0)),
          pl.BlockSpec((_BLK, 128), lambda i, n=nblk: (i + n, 0)),
      ],
      out_specs=pl.BlockSpec((_BLK,), lambda i: (i,)),
      out_shape=jax.ShapeDtypeStruct((BATCH,), jnp.float32),
  )(staging, staging)


def kernel(xs, table):
  t3 = table.T.reshape(4, 8, VOCAB)
  xs2 = xs.reshape(NSLOT // 128, 128)
  staging = _gather(xs2, t3)
  return _compute(staging)


# speedup vs baseline: 3.3825x; 1.3658x over previous
import functools

import jax
import jax.numpy as jnp
from jax import lax
from jax.experimental import pallas as pl
from jax.experimental.pallas import tpu as pltpu
from jax.experimental.pallas import tpu_sc as plsc
from jax._src.pallas.mosaic import sc_core as _sc_core

VOCAB = 1000000
DIM = 32
BATCH = 16384
NSLOT = 2 * BATCH

NC = 2
NS = 16
NW = NC * NS
NTILES = (VOCAB + 127) // 128
TPW = NTILES // NW
TAIL = NTILES - NW * TPW
FL = 48
_RB = 1


def _make_gather_kernel():
  mesh = plsc.VectorSubcoreMesh(core_axis_name="c", subcore_axis_name="s")

  @functools.partial(
      pl.kernel,
      mesh=mesh,
      compiler_params=pltpu.CompilerParams(needs_layout_passes=False),
      out_type=jax.ShapeDtypeStruct((NSLOT, 128), jnp.float32),
      scratch_types=[
          pltpu.VMEM((NSLOT // 128, 128), jnp.int32),
          pltpu.VMEM((NSLOT + 16,), jnp.int32),
          pltpu.VMEM((NSLOT + 16,), jnp.int32),
          pltpu.VMEM((4, 4, 8, 128), jnp.float32),
          pltpu.VMEM((2 * FL, 128), jnp.float32),
          pltpu.VMEM((2 * FL,), jnp.int32),
          pltpu.VMEM((272,), jnp.int32),
          pltpu.VMEM((272,), jnp.int32),
          pltpu.VMEM((272,), jnp.int32),
          pltpu.SemaphoreType.DMA,
          pltpu.SemaphoreType.DMA,
          pltpu.SemaphoreType.DMA,
          pltpu.SemaphoreType.DMA,
          pltpu.SemaphoreType.DMA,
      ],
  )
  def k(xs_hbm, t3_hbm, st_hbm, xsv, hit, srt, ring, exb, exs,
        cntv, curv, st0v, sem0, sem1, semA, semB, sem2):
    wid = lax.axis_index("s") * NC + lax.axis_index("c")
    t0 = wid * TPW
    tcnt = jnp.where(wid == NW - 1, TPW + TAIL, TPW)
    lane_lo = t0 * 128

    lanev = lax.iota(jnp.int32, 16)
    neg16 = jnp.full((16,), -1, jnp.int32)
    slot_lane = (lanev & 1) * BATCH + (lanev >> 1)

    pltpu.sync_copy(xs_hbm, xsv)
    lim = tcnt * 128

    def scan_row(r, off):
      base = r * 128
      for c in range(8):
        idx = xsv[r, pl.ds(c * 16, 16)]
        rl = idx - lane_lo
        m = (rl >= 0) & (rl < lim)
        slot = ((base + c * 16) >> 1) + slot_lane
        packed = (rl << 16) + slot
        plsc.store_compressed(hit.at[pl.ds(off, 16)], packed, mask=m)
        pc = plsc.all_reduce_population_count(m)
        off = off + pc[0]
      return off

    nh = lax.fori_loop(0, NSLOT // 128, scan_row, jnp.int32(0))

    zero16 = jnp.zeros((16,), jnp.int32)
    for i in range(16):
      cntv[pl.ds(i * 16, 16)] = zero16

    ngrp = (nh + 15) >> 4

    def count_grp(g, carry):
      m = g * 16 + lanev < nh
      v = hit[pl.ds(g * 16, 16)]
      t = (v >> 23) & 255
      rank, lastm = plsc.scan_count(t, mask=m)
      plsc.addupdate_scatter(
          cntv, [t], rank - _RB + 1, mask=m & lastm
      )
      return carry

    lax.fori_loop(0, ngrp, count_grp, 0)

    carry = jnp.int32(0)
    for i in range(16):
      cv = cntv[pl.ds(i * 16, 16)]
      inc = plsc.cumsum(cv) + carry
      st0v[pl.ds(i * 16, 16)] = inc - cv
      curv[pl.ds(i * 16, 16)] = inc - cv
      carry = inc[15]

    def place_grp(g, carry):
      m = g * 16 + lanev < nh
      v = hit[pl.ds(g * 16, 16)]
      t = (v >> 23) & 255
      rank, lastm = plsc.scan_count(t, mask=m)
      base = plsc.load_gather(curv, [t])
      pos = base + (rank - _RB)
      plsc.store_scatter(srt, [pos], v, mask=m)
      plsc.addupdate_scatter(
          curv, [t], rank - _RB + 1, mask=m & lastm
      )
      return carry

    lax.fori_loop(0, ngrp, place_grp, 0)

    sems = (sem0, sem1, semA, semB)

    def fire(c, slot):
      start = lane_lo + c * 128
      start = pl.multiple_of(start, 128)
      for i in range(4):
        pltpu.async_copy(
            t3_hbm.at[i, :, pl.ds(start, 128)], ring.at[slot, i], sems[slot]
        )

    def drain(slot):
      for i in range(4):
        pltpu.make_async_copy(
            t3_hbm.at[i, :, pl.ds(0, 128)], ring.at[slot, i], sems[slot]
        ).wait()

    for i in range(2 * FL // 16):
      exs[pl.ds(i * 16, 16)] = neg16

    def wait_one_flush():
      pltpu.make_async_copy(
          exb.at[pl.ds(0, FL), :], st_hbm.at[pl.ds(0, FL)], sem2
      ).wait()

    def fire_flush(half):
      base = half * FL
      pltpu.async_copy(
          exb.at[pl.ds(base, FL), :],
          st_hbm.at[_sc_core.Indices(
              exs.at[pl.ds(base, FL)], ignored_value=-1)],
          sem2,
      )

    def reset_half(half):
      base = half * FL
      for i in range(FL // 16):
        exs[pl.ds(base + i * 16, 16)] = neg16

    dlo = lanev // 8
    dls = lanev % 8
    dhi = 2 + dlo

    def make_process(slot):
      slotv = jnp.full((16,), slot, jnp.int32)

      def process(c, ex):
        k0 = st0v[pl.ds(c, 16)][0]
        kcnt = cntv[pl.ds(c, 16)][0]

        def grp(g, ex):
          n = jnp.minimum(kcnt - g * 16, 16)
          m = lanev < n
          v = srt[pl.ds(k0 + g * 16, 16)]
          l = (v >> 16) & 127
          s = v & 0xFFFF
          erv = lax.rem(ex + lanev, jnp.int32(2 * FL))
          r = lax.rem(ex, jnp.int32(FL))
          cross = (r + n) >= FL
          half_done = lax.rem(ex // FL, jnp.int32(2))
          ex_new = ex + n
          pl.when(cross & (ex_new >= 2 * FL))(wait_one_flush)
          pl.when(cross & (half_done == 1))(lambda: reset_half(0))
          pl.when(cross & (half_done == 0))(lambda: reset_half(1))
          for d in range(DIM):
            gv = plsc.load_gather(
                ring,
                [slotv, jnp.full((16,), d // 8, jnp.int32),
                 jnp.full((16,), d % 8, jnp.int32), l],
            )
            plsc.store_scatter(
                exb, [erv, jnp.full((16,), d, jnp.int32)], gv, mask=m
            )
          n_pre = jnp.minimum(n, FL - r)
          m_pre = lanev < n_pre
          m_post = m & ~m_pre
          plsc.store_scatter(exs, [erv], s, mask=m_pre)
          pl.when(cross & (half_done == 0))(lambda: fire_flush(0))
          pl.when(cross & (half_done == 1))(lambda: fire_flush(1))
          plsc.store_scatter(exs, [erv], s, mask=m_post)
          return ex_new

        return lax.fori_loop(0, (kcnt + 15) >> 4, grp, ex)

      return process

    procs = [make_process(s) for s in range(4)]

    def phase(c, ph, ex):
      pl.when(c + 3 < tcnt)(lambda: fire(c + 3, (ph + 3) % 4))
      drain(ph)
      return procs[ph](c, ex)

    for s in range(3):
      fire(s, s)

    def step4(c4, ex):
      c = c4 * 4
      for ph in range(4):
        ex = phase(c + ph, ph, ex)
      return ex

    ex = lax.fori_loop(0, TPW // 4, step4, jnp.int32(0))

    def tail_steps():
      e = ex
      for c in range(TPW, TPW + TAIL):
        e = phase(c, c % 4, e)
      return e

    ex = lax.cond(wid == NW - 1, tail_steps, lambda: ex)

    nfull = ex // FL
    partial = lax.rem(ex, jnp.int32(FL)) != 0
    ph = lax.rem(nfull, jnp.int32(2))
    pl.when(nfull >= 1)(wait_one_flush)
    pl.when(partial & (ph == 0))(lambda: fire_flush(0))
    pl.when(partial & (ph == 1))(lambda: fire_flush(1))
    pl.when(partial)(wait_one_flush)

  return k


_gather = _make_gather_kernel()


def _compute_body(a_ref, b_ref, o_ref):
  a = a_ref[...]
  b = b_ref[...]
  mask = lax.broadcasted_iota(jnp.int32, a.shape, 1) < DIM
  zero = jnp.zeros_like(a)
  dot = jnp.sum(jnp.where(mask, a * b, zero), axis=1)
  na = jnp.sum(jnp.where(mask, a * a, zero), axis=1)
  nb = jnp.sum(jnp.where(mask, b * b, zero), axis=1)
  cos = dot * lax.rsqrt(na * nb)
  sig = 1.0 / (1.0 + jnp.exp(-cos))
  o_ref[...] = (sig + 1.0) * 0.5


_BLK = 4096


def _compute(staging):
  nblk = BATCH // _BLK
  return pl.pallas_call(
      _compute_body,
      grid=(nblk,),
      in_specs=[
          pl.BlockSpec((_BLK, 128), lambda i: (i, 0)),
          pl.BlockSpec((_BLK, 128), lambda i, n=nblk: (i + n, 0)),
      ],
      out_specs=pl.BlockSpec((_BLK,), lambda i: (i,)),
      out_shape=jax.ShapeDtypeStruct((BATCH,), jnp.float32),
  )(staging, staging)


def kernel(xs, table):
  t3 = table.T.reshape(4, 8, VOCAB)
  xs2 = xs.reshape(NSLOT // 128, 128)
  staging = _gather(xs2, t3)
  return _compute(staging)


# speedup vs baseline: 3.4069x; 1.0072x over previous
import functools

import jax
import jax.numpy as jnp
from jax import lax
from jax.experimental import pallas as pl
from jax.experimental.pallas import tpu as pltpu
from jax.experimental.pallas import tpu_sc as plsc
from jax._src.pallas.mosaic import sc_core as _sc_core

VOCAB = 1000000
DIM = 32
BATCH = 16384
NSLOT = 2 * BATCH

NC = 2
NS = 16
NW = NC * NS
NTILES = (VOCAB + 127) // 128
TPW = NTILES // NW
TAIL = NTILES - NW * TPW
FL = 48
_RB = 1


def _make_gather_kernel():
  mesh = plsc.VectorSubcoreMesh(core_axis_name="c", subcore_axis_name="s")

  @functools.partial(
      pl.kernel,
      mesh=mesh,
      compiler_params=pltpu.CompilerParams(needs_layout_passes=False),
      out_type=jax.ShapeDtypeStruct((NSLOT, 128), jnp.float32),
      scratch_types=[
          pltpu.VMEM((NSLOT // 128, 128), jnp.int32),
          pltpu.VMEM((NSLOT + 16,), jnp.int32),
          pltpu.VMEM((NSLOT + 16,), jnp.int32),
          pltpu.VMEM((4, 4, 8, 128), jnp.float32),
          pltpu.VMEM((2 * FL, 128), jnp.float32),
          pltpu.VMEM((2 * FL,), jnp.int32),
          pltpu.VMEM((272,), jnp.int32),
          pltpu.VMEM((272,), jnp.int32),
          pltpu.VMEM((272,), jnp.int32),
          pltpu.SemaphoreType.DMA,
          pltpu.SemaphoreType.DMA,
          pltpu.SemaphoreType.DMA,
          pltpu.SemaphoreType.DMA,
          pltpu.SemaphoreType.DMA,
      ],
  )
  def k(xs_hbm, t3_hbm, st_hbm, xsv, hit, srt, ring, exb, exs,
        cntv, curv, st0v, sem0, sem1, semA, semB, sem2):
    wid = lax.axis_index("s") * NC + lax.axis_index("c")
    t0 = wid * TPW
    tcnt = jnp.where(wid == NW - 1, TPW + TAIL, TPW)
    lane_lo = t0 * 128

    lanev = lax.iota(jnp.int32, 16)
    neg16 = jnp.full((16,), -1, jnp.int32)
    slot_lane = (lanev & 1) * BATCH + (lanev >> 1)

    sems = (sem0, sem1, semA, semB)

    def fire(c, slot):
      start = lane_lo + c * 128
      start = pl.multiple_of(start, 128)
      for i in range(4):
        pltpu.async_copy(
            t3_hbm.at[i, :, pl.ds(start, 128)], ring.at[slot, i], sems[slot]
        )

    def drain(slot):
      for i in range(4):
        pltpu.make_async_copy(
            t3_hbm.at[i, :, pl.ds(0, 128)], ring.at[slot, i], sems[slot]
        ).wait()

    for s in range(3):
      fire(s, s)

    pltpu.sync_copy(xs_hbm, xsv)
    lim = tcnt * 128

    def scan_row(r, off):
      base = r * 128
      for c in range(8):
        idx = xsv[r, pl.ds(c * 16, 16)]
        rl = idx - lane_lo
        m = (rl >= 0) & (rl < lim)
        slot = ((base + c * 16) >> 1) + slot_lane
        packed = (rl << 16) + slot
        plsc.store_compressed(hit.at[pl.ds(off, 16)], packed, mask=m)
        pc = plsc.all_reduce_population_count(m)
        off = off + pc[0]
      return off

    nh = lax.fori_loop(0, NSLOT // 128, scan_row, jnp.int32(0))

    zero16 = jnp.zeros((16,), jnp.int32)
    for i in range(16):
      cntv[pl.ds(i * 16, 16)] = zero16

    ngrp = (nh + 15) >> 4

    def count_grp(g, carry):
      m = g * 16 + lanev < nh
      v = hit[pl.ds(g * 16, 16)]
      t = (v >> 23) & 255
      rank, lastm = plsc.scan_count(t, mask=m)
      plsc.addupdate_scatter(
          cntv, [t], rank - _RB + 1, mask=m & lastm
      )
      return carry

    lax.fori_loop(0, ngrp, count_grp, 0)

    carry = jnp.int32(0)
    for i in range(16):
      cv = cntv[pl.ds(i * 16, 16)]
      inc = plsc.cumsum(cv) + carry
      st0v[pl.ds(i * 16, 16)] = inc - cv
      curv[pl.ds(i * 16, 16)] = inc - cv
      carry = inc[15]

    def place_grp(g, carry):
      m = g * 16 + lanev < nh
      v = hit[pl.ds(g * 16, 16)]
      t = (v >> 23) & 255
      rank, lastm = plsc.scan_count(t, mask=m)
      base = plsc.load_gather(curv, [t])
      pos = base + (rank - _RB)
      plsc.store_scatter(srt, [pos], v, mask=m)
      plsc.addupdate_scatter(
          curv, [t], rank - _RB + 1, mask=m & lastm
      )
      return carry

    lax.fori_loop(0, ngrp, place_grp, 0)

    for i in range(2 * FL // 16):
      exs[pl.ds(i * 16, 16)] = neg16

    def wait_one_flush():
      pltpu.make_async_copy(
          exb.at[pl.ds(0, FL), :], st_hbm.at[pl.ds(0, FL)], sem2
      ).wait()

    def fire_flush(half):
      base = half * FL
      pltpu.async_copy(
          exb.at[pl.ds(base, FL), :],
          st_hbm.at[_sc_core.Indices(
              exs.at[pl.ds(base, FL)], ignored_value=-1)],
          sem2,
      )

    def reset_half(half):
      base = half * FL
      for i in range(FL // 16):
        exs[pl.ds(base + i * 16, 16)] = neg16

    dlo = lanev // 8
    dls = lanev % 8
    dhi = 2 + dlo

    def make_process(slot):
      slotv = jnp.full((16,), slot, jnp.int32)

      def process(c, ex):
        k0 = st0v[pl.ds(c, 16)][0]
        kcnt = cntv[pl.ds(c, 16)][0]

        def grp(g, ex):
          n = jnp.minimum(kcnt - g * 16, 16)
          m = lanev < n
          v = srt[pl.ds(k0 + g * 16, 16)]
          l = (v >> 16) & 127
          s = v & 0xFFFF
          erv = lax.rem(ex + lanev, jnp.int32(2 * FL))
          r = lax.rem(ex, jnp.int32(FL))
          cross = (r + n) >= FL
          half_done = lax.rem(ex // FL, jnp.int32(2))
          ex_new = ex + n
          pl.when(cross & (ex_new >= 2 * FL))(wait_one_flush)
          pl.when(cross & (half_done == 1))(lambda: reset_half(0))
          pl.when(cross & (half_done == 0))(lambda: reset_half(1))
          for d in range(DIM):
            gv = plsc.load_gather(
                ring,
                [slotv, jnp.full((16,), d // 8, jnp.int32),
                 jnp.full((16,), d % 8, jnp.int32), l],
            )
            plsc.store_scatter(
                exb, [erv, jnp.full((16,), d, jnp.int32)], gv, mask=m
            )
          n_pre = jnp.minimum(n, FL - r)
          m_pre = lanev < n_pre
          m_post = m & ~m_pre
          plsc.store_scatter(exs, [erv], s, mask=m_pre)
          pl.when(cross & (half_done == 0))(lambda: fire_flush(0))
          pl.when(cross & (half_done == 1))(lambda: fire_flush(1))
          plsc.store_scatter(exs, [erv], s, mask=m_post)
          return ex_new

        return lax.fori_loop(0, (kcnt + 15) >> 4, grp, ex)

      return process

    procs = [make_process(s) for s in range(4)]

    def phase(c, ph, ex):
      pl.when(c + 3 < tcnt)(lambda: fire(c + 3, (ph + 3) % 4))
      drain(ph)
      return procs[ph](c, ex)

    def step4(c4, ex):
      c = c4 * 4
      for ph in range(4):
        ex = phase(c + ph, ph, ex)
      return ex

    ex = lax.fori_loop(0, TPW // 4, step4, jnp.int32(0))

    def tail_steps():
      e = ex
      for c in range(TPW, TPW + TAIL):
        e = phase(c, c % 4, e)
      return e

    ex = lax.cond(wid == NW - 1, tail_steps, lambda: ex)

    nfull = ex // FL
    partial = lax.rem(ex, jnp.int32(FL)) != 0
    ph = lax.rem(nfull, jnp.int32(2))
    pl.when(nfull >= 1)(wait_one_flush)
    pl.when(partial & (ph == 0))(lambda: fire_flush(0))
    pl.when(partial & (ph == 1))(lambda: fire_flush(1))
    pl.when(partial)(wait_one_flush)

  return k


_gather = _make_gather_kernel()


def _compute_body(a_ref, b_ref, o_ref):
  a = a_ref[...]
  b = b_ref[...]
  mask = lax.broadcasted_iota(jnp.int32, a.shape, 1) < DIM
  zero = jnp.zeros_like(a)
  dot = jnp.sum(jnp.where(mask, a * b, zero), axis=1)
  na = jnp.sum(jnp.where(mask, a * a, zero), axis=1)
  nb = jnp.sum(jnp.where(mask, b * b, zero), axis=1)
  cos = dot * lax.rsqrt(na * nb)
  sig = 1.0 / (1.0 + jnp.exp(-cos))
  o_ref[...] = (sig + 1.0) * 0.5


_BLK = 4096


def _compute(staging):
  nblk = BATCH // _BLK
  return pl.pallas_call(
      _compute_body,
      grid=(nblk,),
      in_specs=[
          pl.BlockSpec((_BLK, 128), lambda i: (i, 0)),
          pl.BlockSpec((_BLK, 128), lambda i, n=nblk: (i + n, 0)),
      ],
      out_specs=pl.BlockSpec((_BLK,), lambda i: (i,)),
      out_shape=jax.ShapeDtypeStruct((BATCH,), jnp.float32),
  )(staging, staging)


def kernel(xs, table):
  t3 = table.T.reshape(4, 8, VOCAB)
  xs2 = xs.reshape(NSLOT // 128, 128)
  staging = _gather(xs2, t3)
  return _compute(staging)
